# baseline (speedup 1.0000x reference)
import dataclasses
import functools

import jax
import jax.numpy as jnp
from jax import lax
from jax.experimental import pallas as pl
from jax.experimental.pallas import tpu as pltpu
from jax.experimental.pallas import tpu_sc as plsc

_N = 8192
_K = 4096
_ROWS = 128
_NC = 2
_NS = 16
_NW = _NC * _NS
_RPW = _ROWS // _NW


def kernel(x):
    mesh = plsc.VectorSubcoreMesh(core_axis_name="c", subcore_axis_name="s")
    cp = pltpu.CompilerParams()
    if "needs_layout_passes" in pltpu.CompilerParams.__dataclass_fields__:
        cp = dataclasses.replace(cp, needs_layout_passes=False)

    @functools.partial(
        pl.kernel,
        mesh=mesh,
        compiler_params=cp,
        out_type=jax.ShapeDtypeStruct((_ROWS, _N), jnp.float32),
        scratch_types=[
            pltpu.VMEM((_RPW, _N), jnp.float32),
            pltpu.VMEM((_RPW, _N), jnp.int32),
            pltpu.SemaphoreType.DMA,
        ],
    )
    def sc_kernel(x_hbm, o_hbm, xv, kv, sem):
        int_max = jnp.int32(2**31 - 1)
        int_min = jnp.int32(-(2**31))
        kf = jnp.float32(_K)

        wid = lax.axis_index("s") * _NC + lax.axis_index("c")
        base = wid * _RPW
        pltpu.async_copy(x_hbm.at[pl.ds(base, _RPW)], xv, sem).wait()

        @pl.loop(0, _RPW)
        def _(r):
            @pl.loop(0, _N, step=16)
            def _(c):
                xs = xv[r, pl.ds(c, 16)] + 0.0
                b = lax.bitcast_convert_type(xs, jnp.int32)
                kv[r, pl.ds(c, 16)] = jnp.where(b >= 0, b, int_max - b)

        def count_ge(r, cand):
            def cbody(c, acc):
                ks = kv[r, pl.ds(c * 16, 16)]
                return acc + jnp.where(ks >= cand,
                                       jnp.float32(1.0), jnp.float32(0.0))
            acc = lax.fori_loop(0, _N // 16, cbody,
                                jnp.zeros((16,), jnp.float32))
            return jnp.sum(acc)

        @pl.loop(0, _RPW)
        def _(r):
            cnt_pos = count_ge(r, jnp.int32(0))
            pos = cnt_pos >= kf
            prefix0 = jnp.where(pos, jnp.int32(0), int_min)
            cntp0 = jnp.where(pos, cnt_pos, jnp.float32(_N))

            def cond(st):
                return jnp.logical_and(st[0] < 31, st[2] > kf)

            def body(st):
                i, p, cp = st
                bit = jnp.left_shift(jnp.int32(1), jnp.int32(30) - i)
                cand = p + bit
                cnt = count_ge(r, cand)
                take = cnt >= kf
                return (i + jnp.int32(1),
                        jnp.where(take, cand, p),
                        jnp.where(take, cnt, cp))

            _, p, _ = lax.while_loop(
                cond, body, (jnp.int32(0), prefix0, cntp0))

            @pl.loop(0, _N, step=16)
            def _(c):
                ks = kv[r, pl.ds(c, 16)]
                xs = xv[r, pl.ds(c, 16)]
                xv[r, pl.ds(c, 16)] = jnp.where(
                    ks >= p, xs, jnp.float32(0.0))

        pltpu.async_copy(xv, o_hbm.at[pl.ds(base, _RPW)], sem).wait()

    return sc_kernel(x)


# speedup vs baseline: 15.4003x; 15.4003x over previous
import jax
import jax.numpy as jnp
from jax.experimental import pallas as pl
from jax.experimental.pallas import tpu as pltpu

_N = 8192
_K = 4096
_ROWS = 128
_GROUP_ROWS = 32
_GROUPS_PER_BLOCK = 2
_BLOCK_ROWS = _GROUP_ROWS * _GROUPS_PER_BLOCK


def _count_ge32(key, cand):
    one = jnp.float32(1.0)
    zero = jnp.float32(0.0)
    accs = [None] * 4
    for t in range(key.shape[1] // (4 * 128)):
        for j in range(4):
            s = (t * 4 + j) * 128
            part = jnp.where(key[:, s:s + 128] >= cand, one, zero)
            accs[j] = part if accs[j] is None else accs[j] + part
    while len(accs) > 1:
        accs = [a + b for a, b in zip(accs[::2], accs[1::2])]
    return jnp.sum(accs[0], axis=1, keepdims=True)


def _count_ge16(xb, candf):
    one = jnp.bfloat16(1.0)
    zero = jnp.bfloat16(0.0)
    accs = [None] * 4
    for t in range(xb.shape[1] // (4 * 256)):
        for j in range(4):
            s = (t * 4 + j) * 256
            part = jnp.where(xb[:, s:s + 256] >= candf, one, zero)
            accs[j] = part if accs[j] is None else accs[j] + part
    while len(accs) > 1:
        accs = [a + b for a, b in zip(accs[::2], accs[1::2])]
    return jnp.sum(accs[0].astype(jnp.float32), axis=1, keepdims=True)


def _bf16_of_prefix(cand32):
    m = jax.lax.shift_right_arithmetic(cand32, 16)
    vb = jnp.where(m >= 0, m, jnp.int32(32767) - m)
    f = jax.lax.bitcast_convert_type(
        jax.lax.shift_left(vb, 16), jnp.float32)
    return f.astype(jnp.bfloat16)


def _sdd_block(x_ref, o_ref):
    int_max = jnp.int32(2**31 - 1)
    int_min = jnp.int32(-(2**31))
    kf = jnp.float32(_K)
    g_rows = _GROUP_ROWS
    n_groups = _GROUPS_PER_BLOCK

    keys = []
    xbs = []
    xs = []
    for g in range(n_groups):
        x = x_ref[g * g_rows:(g + 1) * g_rows, :]
        xz = x + 0.0
        b = jax.lax.bitcast_convert_type(xz, jnp.int32)
        keys.append(jnp.where(b >= 0, b, int_max - b))
        xbs.append(jax.lax.bitcast_convert_type(
            b & jnp.int32(-65536), jnp.float32).astype(jnp.bfloat16))
        xs.append(x)

    prefixes = []
    cntps = []
    for g in range(n_groups):
        cnt_pos = _count_ge16(xbs[g], jnp.zeros_like(xbs[g][:, :1]))
        pos = cnt_pos >= kf
        prefixes.append(jnp.where(pos, jnp.int32(0), int_min))
        cntps.append(jnp.where(pos, cnt_pos, jnp.float32(_N)))

    def sweep16(i, prefix, cntp, xb):
        bit = jnp.left_shift(jnp.int32(1), jnp.int32(30) - i)
        cand = prefix + bit
        cnt = _count_ge16(xb, _bf16_of_prefix(cand))
        take = cnt >= kf
        return jnp.where(take, cand, prefix), jnp.where(take, cnt, cntp)

    def sweep32(i, prefix, cntp, key):
        bit = jnp.left_shift(jnp.int32(1), jnp.int32(30) - i)
        cand = prefix + bit
        cnt = _count_ge32(key, cand)
        take = cnt >= kf
        return jnp.where(take, cand, prefix), jnp.where(take, cnt, cntp)

    def unpack(state):
        return list(state[1::2]), list(state[2::2])

    def pack(i, ps, cs):
        out = [i]
        for p, c in zip(ps, cs):
            out.extend((p, c))
        return tuple(out)

    def cond_until(bound):
        def cond(state):
            done = jnp.bool_(False)
            for c in state[2::2]:
                done = jnp.logical_or(done, jnp.any(c > kf))
            return jnp.logical_and(state[0] < bound, done)
        return cond

    def body_of(sweep_fn, datas):
        def body(state):
            i = state[0]
            ps, cs = unpack(state)
            for j in range(4):
                for g in range(n_groups):
                    ps[g], cs[g] = sweep_fn(
                        i + jnp.int32(j), ps[g], cs[g], datas[g])
            return pack(i + jnp.int32(4), ps, cs)
        return body

    ps, cs = prefixes, cntps
    for j in range(15):
        for g in range(n_groups):
            ps[g], cs[g] = sweep16(jnp.int32(j), ps[g], cs[g], xbs[g])

    state = jax.lax.while_loop(
        cond_until(31), body_of(sweep32, keys),
        pack(jnp.int32(15), ps, cs))
    ps, cs = unpack(state)

    for g in range(n_groups):
        mask = keys[g] >= ps[g]
        o_ref[g * g_rows:(g + 1) * g_rows, :] = jnp.where(
            mask, xs[g], jnp.float32(0.0))


def kernel(x):
    return pl.pallas_call(
        _sdd_block,
        out_shape=jax.ShapeDtypeStruct(x.shape, x.dtype),
        grid=(_ROWS // _BLOCK_ROWS,),
        in_specs=[pl.BlockSpec((_BLOCK_ROWS, _N), lambda i: (i, 0))],
        out_specs=pl.BlockSpec((_BLOCK_ROWS, _N), lambda i: (i, 0)),
        compiler_params=pltpu.CompilerParams(
            dimension_semantics=("parallel",)
        ),
    )(x)


# speedup vs baseline: 16.6128x; 1.0787x over previous
import jax
import jax.numpy as jnp
from jax.experimental import pallas as pl
from jax.experimental.pallas import tpu as pltpu

_N = 8192
_K = 4096
_ROWS = 128
_GROUP_ROWS = 32
_GROUPS_PER_BLOCK = 2
_BLOCK_ROWS = _GROUP_ROWS * _GROUPS_PER_BLOCK


def _count_ge32(x, candf):
    one = jnp.float32(1.0)
    zero = jnp.float32(0.0)
    accs = [None] * 4
    for t in range(x.shape[1] // (4 * 128)):
        for j in range(4):
            s = (t * 4 + j) * 128
            part = jnp.where(x[:, s:s + 128] >= candf, one, zero)
            accs[j] = part if accs[j] is None else accs[j] + part
    while len(accs) > 1:
        accs = [a + b for a, b in zip(accs[::2], accs[1::2])]
    return jnp.sum(accs[0], axis=1, keepdims=True)


def _count_ge16(xb, candf):
    one = jnp.bfloat16(1.0)
    zero = jnp.bfloat16(0.0)
    accs = [None] * 4
    for t in range(xb.shape[1] // (4 * 256)):
        for j in range(4):
            s = (t * 4 + j) * 256
            part = jnp.where(xb[:, s:s + 256] >= candf, one, zero)
            accs[j] = part if accs[j] is None else accs[j] + part
    while len(accs) > 1:
        accs = [a + b for a, b in zip(accs[::2], accs[1::2])]
    return jnp.sum(accs[0].astype(jnp.float32), axis=1, keepdims=True)


def _bf16_of_prefix(cand32):
    m = jax.lax.shift_right_arithmetic(cand32, 16)
    vb = jnp.where(m >= 0, m, jnp.int32(32767) - m)
    f = jax.lax.bitcast_convert_type(
        jax.lax.shift_left(vb, 16), jnp.float32)
    return f.astype(jnp.bfloat16)


def _f32_of_key(cand32):
    vb = jnp.where(cand32 >= 0, cand32, jnp.int32(2**31 - 1) - cand32)
    return jax.lax.bitcast_convert_type(vb, jnp.float32)


def _sdd_block(x_ref, o_ref):
    int_max = jnp.int32(2**31 - 1)
    int_min = jnp.int32(-(2**31))
    kf = jnp.float32(_K)
    g_rows = _GROUP_ROWS
    n_groups = _GROUPS_PER_BLOCK

    xbs = []
    xs = []
    for g in range(n_groups):
        x = x_ref[g * g_rows:(g + 1) * g_rows, :]
        b = jax.lax.bitcast_convert_type(x, jnp.int32)
        xbs.append(jax.lax.bitcast_convert_type(
            b & jnp.int32(-65536), jnp.float32).astype(jnp.bfloat16))
        xs.append(x)

    prefixes = []
    cntps = []
    for g in range(n_groups):
        cnt_pos = _count_ge16(xbs[g], jnp.zeros_like(xbs[g][:, :1]))
        pos = cnt_pos >= kf
        prefixes.append(jnp.where(pos, jnp.int32(0), int_min))
        cntps.append(jnp.where(pos, cnt_pos, jnp.float32(_N)))

    def sweep16(i, prefix, cntp, xb):
        bit = jnp.left_shift(jnp.int32(1), jnp.int32(30) - i)
        cand = prefix + bit
        cnt = _count_ge16(xb, _bf16_of_prefix(cand))
        take = cnt >= kf
        return jnp.where(take, cand, prefix), jnp.where(take, cnt, cntp)

    def sweep32(i, prefix, cntp, x):
        bit = jnp.left_shift(jnp.int32(1), jnp.int32(30) - i)
        cand = prefix + bit
        cnt = _count_ge32(x, _f32_of_key(cand))
        take = cnt >= kf
        return jnp.where(take, cand, prefix), jnp.where(take, cnt, cntp)

    def unpack(state):
        return list(state[1::2]), list(state[2::2])

    def pack(i, ps, cs):
        out = [i]
        for p, c in zip(ps, cs):
            out.extend((p, c))
        return tuple(out)

    def cond_until(bound):
        def cond(state):
            done = jnp.bool_(False)
            for c in state[2::2]:
                done = jnp.logical_or(done, jnp.any(c > kf))
            return jnp.logical_and(state[0] < bound, done)
        return cond

    def body_of(sweep_fn, datas):
        def body(state):
            i = state[0]
            ps, cs = unpack(state)
            for j in range(4):
                for g in range(n_groups):
                    ps[g], cs[g] = sweep_fn(
                        i + jnp.int32(j), ps[g], cs[g], datas[g])
            return pack(i + jnp.int32(4), ps, cs)
        return body

    ps, cs = prefixes, cntps
    for j in range(15):
        for g in range(n_groups):
            ps[g], cs[g] = sweep16(jnp.int32(j), ps[g], cs[g], xbs[g])

    state = jax.lax.while_loop(
        cond_until(31), body_of(sweep32, xs),
        pack(jnp.int32(15), ps, cs))
    ps, cs = unpack(state)

    for g in range(n_groups):
        mask = xs[g] >= _f32_of_key(ps[g])
        o_ref[g * g_rows:(g + 1) * g_rows, :] = jnp.where(
            mask, xs[g], jnp.float32(0.0))


def kernel(x):
    return pl.pallas_call(
        _sdd_block,
        out_shape=jax.ShapeDtypeStruct(x.shape, x.dtype),
        grid=(_ROWS // _BLOCK_ROWS,),
        in_specs=[pl.BlockSpec((_BLOCK_ROWS, _N), lambda i: (i, 0))],
        out_specs=pl.BlockSpec((_BLOCK_ROWS, _N), lambda i: (i, 0)),
        compiler_params=pltpu.CompilerParams(
            dimension_semantics=("parallel",)
        ),
    )(x)
